# baseline (speedup 1.0000x reference)
import functools

import jax
import jax.numpy as jnp
import numpy as np
from jax import lax
from jax.experimental import pallas as pl
from jax.experimental.pallas import tpu as pltpu
from jax.experimental.pallas import tpu_sc as plsc

N = 10000
NP = 10240
OUTC = 10
F = 128
H = 16
NC = 2
NS = 16
LANES = 16

CH = 1296
GPT = 8
EW = CH * GPT
EP = NC * NS * EW
RPT = NP // NS
DPT = NP // (NC * NS)
DEGR = EP // (NS * CH)

_mesh = plsc.VectorSubcoreMesh(core_axis_name="c", subcore_axis_name="s")


NBUF = 2


@functools.partial(
    pl.kernel,
    out_type=[
        jax.ShapeDtypeStruct((NC, NP, H), jnp.float32),
        jax.ShapeDtypeStruct((NP, H), jnp.float32),
    ],
    mesh=_mesh,
    scratch_types=[
        pltpu.VMEM((GPT, CH), jnp.int32),
        pltpu.VMEM((GPT, CH), jnp.int32),
        pltpu.VMEM((DEGR, CH), jnp.int32),
        pltpu.VMEM((CH,), jnp.float32),
        pltpu.VMEM((RPT,), jnp.float32),
        pltpu.VMEM((RPT, H), jnp.float32),
        pltpu.VMEM((RPT, H), jnp.float32),
        [pltpu.VMEM((CH, H), jnp.float32)] * NBUF,
        [pltpu.SemaphoreType.DMA] * NBUF,
        [pltpu.SemaphoreType.DMA] * NBUF,
        pltpu.SemaphoreType.DMA,
        pltpu.VMEM_SHARED((NP,), jnp.float32),
        pltpu.VMEM_SHARED((NP, H), jnp.float32),
        pltpu.VMEM_SHARED((NP, H), jnp.float32),
    ],
    compiler_params=pltpu.CompilerParams(use_tc_tiling_on_sc=False),
)
def _sc1(h1_hbm, edge_hbm, out_hbm, dis_hbm,
         srcv, dstv, dstdeg, ones, degb, h1v, disb, bufs, gsems, ssems,
         dsem, deg_sh, hp_sh, acc_sh):
    c = lax.axis_index("c")
    s = lax.axis_index("s")
    rowbase = s * RPT

    def zrow(i, _):
        for u in range(4):
            bufs[0][i * 4 + u] = jnp.zeros((H,), jnp.float32)
        return 0

    lax.fori_loop(0, RPT // 4, zrow, 0)

    def zb_init(i, _):
        degb[pl.ds(i * LANES, LANES)] = jnp.zeros((LANES,), jnp.float32)
        return 0

    lax.fori_loop(0, RPT // LANES, zb_init, 0)

    def ones_init(i, _):
        ones[pl.ds(i * LANES, LANES)] = jnp.full((LANES,), 1.0, jnp.float32)
        return 0

    lax.fori_loop(0, CH // LANES, ones_init, 0)
    pltpu.sync_copy(bufs[0].at[pl.ds(0, RPT)], acc_sh.at[pl.ds(rowbase, RPT)])
    pltpu.sync_copy(degb, deg_sh.at[pl.ds(rowbase, RPT)])
    pltpu.sync_copy(h1_hbm.at[pl.ds(rowbase, RPT)], h1v)
    wid = c * NS + s
    pltpu.sync_copy(edge_hbm.at[pl.ds(s * DEGR, DEGR)], dstdeg)
    pltpu.sync_copy(edge_hbm.at[pl.ds(wid * GPT, GPT)], srcv)

    for r in range(DEGR):
        def unpk_d(j, _, r=r):
            for u in range(3):
                sl = pl.ds((j * 3 + u) * LANES, LANES)
                dstdeg[r, sl] = lax.shift_right_arithmetic(dstdeg[r, sl], 16)
            return 0

        lax.fori_loop(0, CH // (3 * LANES), unpk_d, 0)
    for g in range(GPT):
        def unpk(j, _, g=g):
            for u in range(3):
                sl = pl.ds((j * 3 + u) * LANES, LANES)
                v = srcv[g, sl]
                dstv[g, sl] = lax.shift_right_arithmetic(v, 16)
                srcv[g, sl] = v & jnp.int32(0xFFFF)
            return 0

        lax.fori_loop(0, CH // (3 * LANES), unpk, 0)
    plsc.subcore_barrier()

    descs = [pltpu.async_copy(ones, deg_sh.at[dstdeg.at[r]], dsem, add=True)
             for r in range(DEGR)]
    for d in descs:
        d.wait()
    plsc.subcore_barrier()

    pltpu.sync_copy(deg_sh.at[pl.ds(rowbase, RPT)], degb)

    def scale_blk(i, _):
        v = degb[pl.ds(i * LANES, LANES)]
        v = jnp.maximum(v, jnp.float32(1.0))
        bi = lax.bitcast_convert_type(v, jnp.int32)
        bi = jnp.int32(0x5F3759DF) - lax.shift_right_arithmetic(bi, 1)
        y = lax.bitcast_convert_type(bi, jnp.float32)
        for _ in range(3):
            y = y * (jnp.float32(1.5) - jnp.float32(0.5) * v * y * y)
        for k in range(LANES):
            r = i * LANES + k
            dv = jnp.full((LANES,), y[k], jnp.float32)
            disb[r] = dv
            h1v[r] = h1v[r] * dv
        return 0

    lax.fori_loop(0, RPT // LANES, scale_blk, 0)
    pltpu.sync_copy(h1v, hp_sh.at[pl.ds(rowbase, RPT)])

    @pl.when(c == 0)
    def _():
        pltpu.sync_copy(disb, dis_hbm.at[pl.ds(rowbase, RPT)])

    plsc.subcore_barrier()

    gd = [None] * GPT
    sd = [None] * GPT
    gd[0] = pltpu.async_copy(hp_sh.at[srcv.at[0]], bufs[0], gsems[0])
    for g in range(GPT):
        gd[g].wait()
        sd[g] = pltpu.async_copy(bufs[g % NBUF], acc_sh.at[dstv.at[g]],
                                 ssems[g % NBUF], add=True)
        ng = g + 1
        if ng < GPT:
            if ng >= NBUF:
                sd[ng - NBUF].wait()
            gd[ng] = pltpu.async_copy(hp_sh.at[srcv.at[ng]],
                                      bufs[ng % NBUF], gsems[ng % NBUF])
    for g in range(max(0, GPT - NBUF), GPT):
        sd[g].wait()
    plsc.subcore_barrier()
    pltpu.sync_copy(acc_sh.at[pl.ds(rowbase, RPT)],
                    out_hbm.at[c, pl.ds(rowbase, RPT)])


@functools.partial(
    pl.kernel,
    out_type=jax.ShapeDtypeStruct((NC, NP, H), jnp.float32),
    mesh=_mesh,
    scratch_types=[
        pltpu.VMEM((GPT, CH), jnp.int32),
        pltpu.VMEM((GPT, CH), jnp.int32),
        [pltpu.VMEM((CH, H), jnp.float32)] * 4,
        [pltpu.SemaphoreType.DMA] * 4,
        [pltpu.SemaphoreType.DMA] * 4,
        pltpu.VMEM_SHARED((NP, H), jnp.float32),
    ],
    compiler_params=pltpu.CompilerParams(use_tc_tiling_on_sc=False),
)
def _agg(h_hbm, edge_hbm, out_hbm, srcv, dstv, bufs, gsems, ssems,
         acc_sh):
    c = lax.axis_index("c")
    s = lax.axis_index("s")
    rows0 = bufs[0]

    def zrow(i, _):
        for u in range(4):
            rows0[i * 4 + u] = jnp.zeros((H,), jnp.float32)
        return 0

    lax.fori_loop(0, RPT // 4, zrow, 0)
    pltpu.sync_copy(rows0.at[pl.ds(0, RPT)], acc_sh.at[pl.ds(s * RPT, RPT)])
    plsc.subcore_barrier()

    wid = c * NS + s
    pltpu.sync_copy(edge_hbm.at[pl.ds(wid * GPT, GPT)], srcv)
    for g in range(GPT):
        def unpk(j, _, g=g):
            for u in range(3):
                sl = pl.ds((j * 3 + u) * LANES, LANES)
                v = srcv[g, sl]
                dstv[g, sl] = lax.shift_right_arithmetic(v, 16)
                srcv[g, sl] = v & jnp.int32(0xFFFF)
            return 0

        lax.fori_loop(0, CH // (3 * LANES), unpk, 0)

    gd = [None] * GPT
    sd = [None] * GPT
    for g in range(min(2, GPT)):
        gd[g] = pltpu.async_copy(h_hbm.at[srcv.at[g]], bufs[g % 4],
                                 gsems[g % 4])
    for g in range(GPT):
        gd[g].wait()
        sd[g] = pltpu.async_copy(bufs[g % 4], acc_sh.at[dstv.at[g]],
                                 ssems[g % 4], add=True)
        ng = g + 2
        if ng < GPT:
            if ng >= 4:
                sd[ng - 4].wait()
            gd[ng] = pltpu.async_copy(h_hbm.at[srcv.at[ng]], bufs[ng % 4],
                                      gsems[ng % 4])
    for g in range(max(0, GPT - 4), GPT):
        sd[g].wait()
    plsc.subcore_barrier()
    pltpu.sync_copy(acc_sh.at[pl.ds(s * RPT, RPT)],
                    out_hbm.at[c, pl.ds(s * RPT, RPT)])


def _lin1_body(x_ref, w_ref, o_ref):
    h = jnp.dot(x_ref[...], w_ref[...], preferred_element_type=jnp.float32)
    o_ref[...] = jnp.concatenate(
        [h, jnp.zeros((NP - N, H), jnp.float32)], axis=0)


def _lin2_body(p_ref, dis_ref, b1_ref, w2_ref, o_ref):
    acc = p_ref[0] + p_ref[1]
    out1 = jnp.maximum(acc * dis_ref[...] + b1_ref[...], 0.0)
    w2 = jnp.concatenate(
        [w2_ref[...], jnp.zeros((H, H - OUTC), jnp.float32)], axis=1)
    h2 = jnp.dot(out1, w2, preferred_element_type=jnp.float32)
    o_ref[...] = h2 * dis_ref[...]


def _fin_body(p_ref, dis_ref, b2_ref, o_ref):
    out = (p_ref[0] + p_ref[1]) * dis_ref[...]
    o_ref[...] = out[:N, :OUTC] + b2_ref[...]


_LOOP_NP = np.arange(N, dtype=np.int32)
_PAD_NP = (N + np.arange(EP - 320000 - N, dtype=np.int32) % (NP - N))
_TAIL_NP = np.concatenate([_LOOP_NP, _PAD_NP])
_TAIL_PACKED = (_TAIL_NP | (_TAIL_NP << 16)).astype(np.int32)


def kernel(x, edge_index, W1, b1, W2, b2):
    src = edge_index[0].astype(jnp.int32)
    dst = edge_index[1].astype(jnp.int32)
    edges = jnp.concatenate([src | (dst << 16), _TAIL_PACKED])
    edges = edges.reshape(EP // CH, CH)

    h1 = pl.pallas_call(
        _lin1_body,
        out_shape=jax.ShapeDtypeStruct((NP, H), jnp.float32),
    )(x, W1)

    p1, dis = _sc1(h1, edges)

    h2 = pl.pallas_call(
        _lin2_body,
        out_shape=jax.ShapeDtypeStruct((NP, H), jnp.float32),
    )(p1, dis, b1.reshape(1, H), W2)

    p2 = _agg(h2, edges)

    out = pl.pallas_call(
        _fin_body,
        out_shape=jax.ShapeDtypeStruct((N, OUTC), jnp.float32),
    )(p2, dis, b2.reshape(1, OUTC))

    return out


# speedup vs baseline: 1.0016x; 1.0016x over previous
import functools

import jax
import jax.numpy as jnp
import numpy as np
from jax import lax
from jax.experimental import pallas as pl
from jax.experimental.pallas import tpu as pltpu
from jax.experimental.pallas import tpu_sc as plsc

N = 10000
NP = 10240
OUTC = 10
F = 128
H = 16
NC = 2
NS = 16
LANES = 16

CH = 1296
GPT = 8
EW = CH * GPT
EP = NC * NS * EW
RPT = NP // NS
DPT = NP // (NC * NS)
DEGR = EP // (NS * CH)

_mesh = plsc.VectorSubcoreMesh(core_axis_name="c", subcore_axis_name="s")


NBUF = 2


@functools.partial(
    pl.kernel,
    out_type=[
        jax.ShapeDtypeStruct((NC, NP, H), jnp.float32),
        jax.ShapeDtypeStruct((NP, H), jnp.float32),
    ],
    mesh=_mesh,
    scratch_types=[
        pltpu.VMEM((GPT, CH), jnp.int32),
        pltpu.VMEM((GPT, CH), jnp.int32),
        pltpu.VMEM((DEGR, CH), jnp.int32),
        pltpu.VMEM((CH,), jnp.float32),
        pltpu.VMEM((RPT,), jnp.float32),
        pltpu.VMEM((RPT, H), jnp.float32),
        pltpu.VMEM((RPT, H), jnp.float32),
        [pltpu.VMEM((CH, H), jnp.float32)] * NBUF,
        [pltpu.SemaphoreType.DMA] * NBUF,
        [pltpu.SemaphoreType.DMA] * NBUF,
        pltpu.SemaphoreType.DMA,
        pltpu.VMEM_SHARED((NP,), jnp.float32),
        pltpu.VMEM_SHARED((NP, H), jnp.float32),
        pltpu.VMEM_SHARED((NP, H), jnp.float32),
    ],
    compiler_params=pltpu.CompilerParams(use_tc_tiling_on_sc=False, skip_device_barrier=True),
)
def _sc1(h1_hbm, edge_hbm, out_hbm, dis_hbm,
         srcv, dstv, dstdeg, ones, degb, h1v, disb, bufs, gsems, ssems,
         dsem, deg_sh, hp_sh, acc_sh):
    c = lax.axis_index("c")
    s = lax.axis_index("s")
    rowbase = s * RPT

    def zrow(i, _):
        for u in range(4):
            bufs[0][i * 4 + u] = jnp.zeros((H,), jnp.float32)
        return 0

    lax.fori_loop(0, RPT // 4, zrow, 0)

    def zb_init(i, _):
        degb[pl.ds(i * LANES, LANES)] = jnp.zeros((LANES,), jnp.float32)
        return 0

    lax.fori_loop(0, RPT // LANES, zb_init, 0)

    def ones_init(i, _):
        ones[pl.ds(i * LANES, LANES)] = jnp.full((LANES,), 1.0, jnp.float32)
        return 0

    lax.fori_loop(0, CH // LANES, ones_init, 0)
    pltpu.sync_copy(bufs[0].at[pl.ds(0, RPT)], acc_sh.at[pl.ds(rowbase, RPT)])
    pltpu.sync_copy(degb, deg_sh.at[pl.ds(rowbase, RPT)])
    pltpu.sync_copy(h1_hbm.at[pl.ds(rowbase, RPT)], h1v)
    wid = c * NS + s
    pltpu.sync_copy(edge_hbm.at[pl.ds(s * DEGR, DEGR)], dstdeg)
    pltpu.sync_copy(edge_hbm.at[pl.ds(wid * GPT, GPT)], srcv)

    for r in range(DEGR):
        def unpk_d(j, _, r=r):
            for u in range(3):
                sl = pl.ds((j * 3 + u) * LANES, LANES)
                dstdeg[r, sl] = lax.shift_right_arithmetic(dstdeg[r, sl], 16)
            return 0

        lax.fori_loop(0, CH // (3 * LANES), unpk_d, 0)
    for g in range(GPT):
        def unpk(j, _, g=g):
            for u in range(3):
                sl = pl.ds((j * 3 + u) * LANES, LANES)
                v = srcv[g, sl]
                dstv[g, sl] = lax.shift_right_arithmetic(v, 16)
                srcv[g, sl] = v & jnp.int32(0xFFFF)
            return 0

        lax.fori_loop(0, CH // (3 * LANES), unpk, 0)
    plsc.subcore_barrier()

    descs = [pltpu.async_copy(ones, deg_sh.at[dstdeg.at[r]], dsem, add=True)
             for r in range(DEGR)]
    for d in descs:
        d.wait()
    plsc.subcore_barrier()

    pltpu.sync_copy(deg_sh.at[pl.ds(rowbase, RPT)], degb)

    def scale_blk(i, _):
        v = degb[pl.ds(i * LANES, LANES)]
        v = jnp.maximum(v, jnp.float32(1.0))
        bi = lax.bitcast_convert_type(v, jnp.int32)
        bi = jnp.int32(0x5F3759DF) - lax.shift_right_arithmetic(bi, 1)
        y = lax.bitcast_convert_type(bi, jnp.float32)
        for _ in range(3):
            y = y * (jnp.float32(1.5) - jnp.float32(0.5) * v * y * y)
        for k in range(LANES):
            r = i * LANES + k
            dv = jnp.full((LANES,), y[k], jnp.float32)
            disb[r] = dv
            h1v[r] = h1v[r] * dv
        return 0

    lax.fori_loop(0, RPT // LANES, scale_blk, 0)
    pltpu.sync_copy(h1v, hp_sh.at[pl.ds(rowbase, RPT)])

    @pl.when(c == 0)
    def _():
        pltpu.sync_copy(disb, dis_hbm.at[pl.ds(rowbase, RPT)])

    plsc.subcore_barrier()

    gd = [None] * GPT
    sd = [None] * GPT
    gd[0] = pltpu.async_copy(hp_sh.at[srcv.at[0]], bufs[0], gsems[0])
    for g in range(GPT):
        gd[g].wait()
        sd[g] = pltpu.async_copy(bufs[g % NBUF], acc_sh.at[dstv.at[g]],
                                 ssems[g % NBUF], add=True)
        ng = g + 1
        if ng < GPT:
            if ng >= NBUF:
                sd[ng - NBUF].wait()
            gd[ng] = pltpu.async_copy(hp_sh.at[srcv.at[ng]],
                                      bufs[ng % NBUF], gsems[ng % NBUF])
    for g in range(max(0, GPT - NBUF), GPT):
        sd[g].wait()
    plsc.subcore_barrier()
    pltpu.sync_copy(acc_sh.at[pl.ds(rowbase, RPT)],
                    out_hbm.at[c, pl.ds(rowbase, RPT)])


@functools.partial(
    pl.kernel,
    out_type=jax.ShapeDtypeStruct((NC, NP, H), jnp.float32),
    mesh=_mesh,
    scratch_types=[
        pltpu.VMEM((GPT, CH), jnp.int32),
        pltpu.VMEM((GPT, CH), jnp.int32),
        [pltpu.VMEM((CH, H), jnp.float32)] * 4,
        [pltpu.SemaphoreType.DMA] * 4,
        [pltpu.SemaphoreType.DMA] * 4,
        pltpu.VMEM_SHARED((NP, H), jnp.float32),
    ],
    compiler_params=pltpu.CompilerParams(use_tc_tiling_on_sc=False, skip_device_barrier=True),
)
def _agg(h_hbm, edge_hbm, out_hbm, srcv, dstv, bufs, gsems, ssems,
         acc_sh):
    c = lax.axis_index("c")
    s = lax.axis_index("s")
    rows0 = bufs[0]

    def zrow(i, _):
        for u in range(4):
            rows0[i * 4 + u] = jnp.zeros((H,), jnp.float32)
        return 0

    lax.fori_loop(0, RPT // 4, zrow, 0)
    pltpu.sync_copy(rows0.at[pl.ds(0, RPT)], acc_sh.at[pl.ds(s * RPT, RPT)])
    plsc.subcore_barrier()

    wid = c * NS + s
    pltpu.sync_copy(edge_hbm.at[pl.ds(wid * GPT, GPT)], srcv)
    for g in range(GPT):
        def unpk(j, _, g=g):
            for u in range(3):
                sl = pl.ds((j * 3 + u) * LANES, LANES)
                v = srcv[g, sl]
                dstv[g, sl] = lax.shift_right_arithmetic(v, 16)
                srcv[g, sl] = v & jnp.int32(0xFFFF)
            return 0

        lax.fori_loop(0, CH // (3 * LANES), unpk, 0)

    gd = [None] * GPT
    sd = [None] * GPT
    for g in range(min(2, GPT)):
        gd[g] = pltpu.async_copy(h_hbm.at[srcv.at[g]], bufs[g % 4],
                                 gsems[g % 4])
    for g in range(GPT):
        gd[g].wait()
        sd[g] = pltpu.async_copy(bufs[g % 4], acc_sh.at[dstv.at[g]],
                                 ssems[g % 4], add=True)
        ng = g + 2
        if ng < GPT:
            if ng >= 4:
                sd[ng - 4].wait()
            gd[ng] = pltpu.async_copy(h_hbm.at[srcv.at[ng]], bufs[ng % 4],
                                      gsems[ng % 4])
    for g in range(max(0, GPT - 4), GPT):
        sd[g].wait()
    plsc.subcore_barrier()
    pltpu.sync_copy(acc_sh.at[pl.ds(s * RPT, RPT)],
                    out_hbm.at[c, pl.ds(s * RPT, RPT)])


def _lin1_body(x_ref, w_ref, o_ref):
    h = jnp.dot(x_ref[...], w_ref[...], preferred_element_type=jnp.float32)
    o_ref[...] = jnp.concatenate(
        [h, jnp.zeros((NP - N, H), jnp.float32)], axis=0)


def _lin2_body(p_ref, dis_ref, b1_ref, w2_ref, o_ref):
    acc = p_ref[0] + p_ref[1]
    out1 = jnp.maximum(acc * dis_ref[...] + b1_ref[...], 0.0)
    w2 = jnp.concatenate(
        [w2_ref[...], jnp.zeros((H, H - OUTC), jnp.float32)], axis=1)
    h2 = jnp.dot(out1, w2, preferred_element_type=jnp.float32)
    o_ref[...] = h2 * dis_ref[...]


def _fin_body(p_ref, dis_ref, b2_ref, o_ref):
    out = (p_ref[0] + p_ref[1]) * dis_ref[...]
    o_ref[...] = out[:N, :OUTC] + b2_ref[...]


_LOOP_NP = np.arange(N, dtype=np.int32)
_PAD_NP = (N + np.arange(EP - 320000 - N, dtype=np.int32) % (NP - N))
_TAIL_NP = np.concatenate([_LOOP_NP, _PAD_NP])
_TAIL_PACKED = (_TAIL_NP | (_TAIL_NP << 16)).astype(np.int32)


def kernel(x, edge_index, W1, b1, W2, b2):
    src = edge_index[0].astype(jnp.int32)
    dst = edge_index[1].astype(jnp.int32)
    edges = jnp.concatenate([src | (dst << 16), _TAIL_PACKED])
    edges = edges.reshape(EP // CH, CH)

    h1 = pl.pallas_call(
        _lin1_body,
        out_shape=jax.ShapeDtypeStruct((NP, H), jnp.float32),
    )(x, W1)

    p1, dis = _sc1(h1, edges)

    h2 = pl.pallas_call(
        _lin2_body,
        out_shape=jax.ShapeDtypeStruct((NP, H), jnp.float32),
    )(p1, dis, b1.reshape(1, H), W2)

    p2 = _agg(h2, edges)

    out = pl.pallas_call(
        _fin_body,
        out_shape=jax.ShapeDtypeStruct((N, OUTC), jnp.float32),
    )(p2, dis, b2.reshape(1, OUTC))

    return out
